# Initial kernel scaffold; baseline (speedup 1.0000x reference)
import jax
import jax.numpy as jnp
from jax.experimental import pallas as pl


def kernel(X, k):
    raise NotImplementedError("write your pallas kernel here")


# speedup vs baseline: 2.5395x; 2.5395x over previous
import jax
import jax.numpy as jnp
from jax.experimental import pallas as pl
from jax.experimental.pallas import tpu as pltpu

_N = 16384
_D = 64
_K = 32
_BM = 256


def _knn_block_kernel(xb_ref, xt_ref, c_ref, i_ref):
    i = pl.program_id(0)
    xb = xb_ref[...]
    xt = xt_ref[...]
    sq_col = jnp.sum(xt * xt, axis=0, keepdims=True)
    sq_row = jnp.sum(xb * xb, axis=1, keepdims=True)
    d = sq_col - 2.0 * jnp.dot(xb, xt, preferred_element_type=jnp.float32)
    cols = jax.lax.broadcasted_iota(jnp.int32, (_BM, _N), 1)
    rows = jax.lax.broadcasted_iota(jnp.int32, (_BM, _N), 0) + i * _BM
    d = jnp.where(cols == rows, jnp.inf, d)

    kcols = jax.lax.broadcasted_iota(jnp.int32, (_BM, _K), 1)

    def body(j, carry):
        d, cacc, iacc = carry
        m = jnp.min(d, axis=1, keepdims=True)
        idx = jnp.min(jnp.where(d == m, cols, _N), axis=1, keepdims=True)
        d = jnp.where(cols == idx, jnp.inf, d)
        cacc = jnp.where(kcols == j, m, cacc)
        iacc = jnp.where(kcols == j, idx, iacc)
        return d, cacc, iacc

    cacc = jnp.zeros((_BM, _K), jnp.float32)
    iacc = jnp.zeros((_BM, _K), jnp.int32)
    _, cacc, iacc = jax.lax.fori_loop(0, _K, body, (d, cacc, iacc))
    c_ref[...] = cacc + sq_row
    i_ref[...] = iacc


def kernel(X, k):
    del k
    xt = X.T
    grid = (_N // _BM,)
    c, idx = pl.pallas_call(
        _knn_block_kernel,
        grid=grid,
        in_specs=[
            pl.BlockSpec((_BM, _D), lambda i: (i, 0)),
            pl.BlockSpec((_D, _N), lambda i: (0, 0)),
        ],
        out_specs=[
            pl.BlockSpec((_BM, _K), lambda i: (i, 0)),
            pl.BlockSpec((_BM, _K), lambda i: (i, 0)),
        ],
        out_shape=[
            jax.ShapeDtypeStruct((_N, _K), jnp.float32),
            jax.ShapeDtypeStruct((_N, _K), jnp.int32),
        ],
    )(X, xt)
    return c, idx


# speedup vs baseline: 2.8289x; 1.1140x over previous
import jax
import jax.numpy as jnp
from jax.experimental import pallas as pl
from jax.experimental.pallas import tpu as pltpu

_N = 16384
_D = 64
_K = 32
_BM = 128
_CW = 2048
_GW = 128
_NG = _N // _GW
_INF = jnp.inf


def _knn_block_kernel(xb_ref, xt_ref, c_ref, i_ref, dscr_ref):
    i = pl.program_id(0)
    xb = xb_ref[...]
    sq_row = jnp.sum(xb * xb, axis=1, keepdims=True)
    g1v_l, g1i_l, g2v_l, g2i_l, g3v_l, g3i_l = [], [], [], [], [], []
    ng_c = _CW // _GW
    for cc in range(_N // _CW):
        xt_c = xt_ref[:, cc * _CW:(cc + 1) * _CW]
        sq_c = jnp.sum(xt_c * xt_c, axis=0, keepdims=True)
        d = sq_row + sq_c - 2.0 * jnp.dot(
            xb, xt_c, preferred_element_type=jnp.float32)
        colc = jax.lax.broadcasted_iota(jnp.int32, (_BM, _CW), 1) + cc * _CW
        rowc = jax.lax.broadcasted_iota(jnp.int32, (_BM, _CW), 0) + i * _BM
        d = jnp.where(colc == rowc, _INF, d)
        dscr_ref[:, cc * _CW:(cc + 1) * _CW] = d
        d3 = d.reshape(_BM, ng_c, _GW)
        c3 = colc.reshape(_BM, ng_c, _GW)
        a1v = jnp.min(d3, axis=2)
        a1i = jnp.min(jnp.where(d3 == a1v[:, :, None], c3, _N), axis=2)
        m = jnp.where(c3 == a1i[:, :, None], _INF, d3)
        a2v = jnp.min(m, axis=2)
        a2i = jnp.min(jnp.where(m == a2v[:, :, None], c3, _N), axis=2)
        m = jnp.where(c3 == a2i[:, :, None], _INF, m)
        a3v = jnp.min(m, axis=2)
        a3i = jnp.min(jnp.where(m == a3v[:, :, None], c3, _N), axis=2)
        g1v_l.append(a1v)
        g1i_l.append(a1i)
        g2v_l.append(a2v)
        g2i_l.append(a2i)
        g3v_l.append(a3v)
        g3i_l.append(a3i)
    g2v = jnp.concatenate(g2v_l, axis=1)
    g2i = jnp.concatenate(g2i_l, axis=1)
    g3v = jnp.concatenate(g3v_l, axis=1)
    g3i = jnp.concatenate(g3i_l, axis=1)
    cols = jax.lax.broadcasted_iota(jnp.int32, (_BM, _N), 1)

    kcols = jax.lax.broadcasted_iota(jnp.int32, (_BM, _K), 1)
    giota = jax.lax.broadcasted_iota(jnp.int32, (_BM, _NG), 1)

    def body(j, carry):
        wv, wi, cnt, cacc, iacc = carry
        mv = jnp.min(wv, axis=1, keepdims=True)
        hits = wv == mv
        gsel = jnp.min(jnp.where(hits, giota, _NG), axis=1, keepdims=True)
        onehot = hits & (giota == gsel)
        mi = jnp.sum(jnp.where(onehot, wi, 0), axis=1, keepdims=True)
        cnt_g = jnp.sum(jnp.where(onehot, cnt, 0), axis=1, keepdims=True)
        cacc = jnp.where(kcols == j, mv, cacc)
        iacc = jnp.where(kcols == j, mi, iacc)
        nv = jnp.where(cnt_g == 0, g2v, jnp.where(cnt_g == 1, g3v, _INF))
        ni = jnp.where(cnt_g == 0, g2i, jnp.where(cnt_g == 1, g3i, _N))
        wv = jnp.where(onehot, nv, wv)
        wi = jnp.where(onehot, ni, wi)
        cnt = cnt + onehot.astype(jnp.int32)
        need = cnt_g >= 2

        def rescan(ws):
            wv2, wi2 = ws
            g = jnp.min(jnp.where(onehot, giota, _NG), axis=1, keepdims=True)
            colg = jax.lax.shift_right_logical(cols, 7)
            dfull = dscr_ref[...]
            after = (dfull > mv) | ((dfull == mv) & (cols > mi))
            valid = (colg == g) & after
            tv = jnp.min(jnp.where(valid, dfull, _INF), axis=1, keepdims=True)
            ti = jnp.min(jnp.where(valid & (dfull == tv), cols, _N),
                         axis=1, keepdims=True)
            sel = onehot & need
            return jnp.where(sel, tv, wv2), jnp.where(sel, ti, wi2)

        wv, wi = jax.lax.cond(jnp.any(need), rescan, lambda ws: ws, (wv, wi))
        return wv, wi, cnt, cacc, iacc

    wv0 = jnp.concatenate(g1v_l, axis=1)
    wi0 = jnp.concatenate(g1i_l, axis=1)
    cnt0 = jnp.zeros((_BM, _NG), jnp.int32)
    cacc0 = jnp.zeros((_BM, _K), jnp.float32)
    iacc0 = jnp.zeros((_BM, _K), jnp.int32)
    _, _, _, cacc, iacc = jax.lax.fori_loop(
        0, _K, body, (wv0, wi0, cnt0, cacc0, iacc0))
    c_ref[...] = cacc
    i_ref[...] = iacc


def kernel(X, k):
    del k
    xt = X.T
    grid = (_N // _BM,)
    c, idx = pl.pallas_call(
        _knn_block_kernel,
        grid=grid,
        in_specs=[
            pl.BlockSpec((_BM, _D), lambda i: (i, 0)),
            pl.BlockSpec((_D, _N), lambda i: (0, 0)),
        ],
        out_specs=[
            pl.BlockSpec((_BM, _K), lambda i: (i, 0)),
            pl.BlockSpec((_BM, _K), lambda i: (i, 0)),
        ],
        out_shape=[
            jax.ShapeDtypeStruct((_N, _K), jnp.float32),
            jax.ShapeDtypeStruct((_N, _K), jnp.int32),
        ],
        scratch_shapes=[pltpu.VMEM((_BM, _N), jnp.float32)],
    )(X, xt)
    return c, idx


# speedup vs baseline: 5.0916x; 1.7998x over previous
import jax
import jax.numpy as jnp
from jax.experimental import pallas as pl
from jax.experimental.pallas import tpu as pltpu

_N = 16384
_D = 64
_K = 32
_BM = 128
_CW = 2048
_GW = 128
_NG = _N // _GW
_T = 6
_INF = jnp.inf


def _knn_block_kernel(xb_ref, xt_ref, c_ref, i_ref, dscr_ref):
    i = pl.program_id(0)
    xb = xb_ref[...]
    sq_row = jnp.sum(xb * xb, axis=1, keepdims=True)
    gv_l = [[] for _ in range(_T)]
    gi_l = [[] for _ in range(_T)]
    ng_c = _CW // _GW
    for cc in range(_N // _CW):
        xt_c = xt_ref[:, cc * _CW:(cc + 1) * _CW]
        sq_c = jnp.sum(xt_c * xt_c, axis=0, keepdims=True)
        d = sq_row + sq_c - 2.0 * jnp.dot(
            xb, xt_c, preferred_element_type=jnp.float32)
        colc = jax.lax.broadcasted_iota(jnp.int32, (_BM, _CW), 1) + cc * _CW
        rowc = jax.lax.broadcasted_iota(jnp.int32, (_BM, _CW), 0) + i * _BM
        d = jnp.where(colc == rowc, _INF, d)
        dscr_ref[:, cc * _CW:(cc + 1) * _CW] = d
        m = d.reshape(_BM, ng_c, _GW)
        c3 = colc.reshape(_BM, ng_c, _GW)
        for lvl in range(_T):
            av = jnp.min(m, axis=2)
            ai = jnp.min(jnp.where(m == av[:, :, None], c3, _N), axis=2)
            gv_l[lvl].append(av)
            gi_l[lvl].append(ai)
            if lvl + 1 < _T:
                m = jnp.where(c3 == ai[:, :, None], _INF, m)
    w0 = jnp.concatenate([a for lst in gv_l for a in lst], axis=1)
    widx = jnp.concatenate([a for lst in gi_l for a in lst], axis=1)

    kcols = jax.lax.broadcasted_iota(jnp.int32, (_BM, _K), 1)

    def body(j, carry):
        w, cacc, iacc = carry
        mv = jnp.min(w, axis=1, keepdims=True)
        mi = jnp.min(jnp.where(w == mv, widx, _N), axis=1, keepdims=True)
        w = jnp.where(widx == mi, _INF, w)
        cacc = jnp.where(kcols == j, mv, cacc)
        iacc = jnp.where(kcols == j, mi, iacc)
        return w, cacc, iacc

    cacc0 = jnp.zeros((_BM, _K), jnp.float32)
    iacc0 = jnp.zeros((_BM, _K), jnp.int32)
    _, cacc, iacc = jax.lax.fori_loop(0, _K, body, (w0, cacc0, iacc0))

    giota = jax.lax.broadcasted_iota(jnp.int32, (_BM, _K, _NG), 2)
    grp_sel = jax.lax.shift_right_logical(iacc, 7)
    cnt_sel = jnp.sum((grp_sel[:, :, None] == giota).astype(jnp.int32), axis=1)
    frow = jnp.any(cnt_sel >= _T, axis=1, keepdims=True)

    def fallback(carry):
        cacc2, iacc2 = carry

        def fb_body(j, c2):
            ca, ia, pv, pi = c2
            mv = jnp.full((_BM, 1), _INF, jnp.float32)
            mi = jnp.full((_BM, 1), _N, jnp.int32)
            for cc in range(_N // _CW):
                dc = dscr_ref[:, cc * _CW:(cc + 1) * _CW]
                colc = (jax.lax.broadcasted_iota(jnp.int32, (_BM, _CW), 1)
                        + cc * _CW)
                validc = (dc > pv) | ((dc == pv) & (colc > pi))
                tv = jnp.min(jnp.where(validc, dc, _INF),
                             axis=1, keepdims=True)
                ti = jnp.min(jnp.where(validc & (dc == tv), colc, _N),
                             axis=1, keepdims=True)
                better = (tv < mv) | ((tv == mv) & (ti < mi))
                mv = jnp.where(better, tv, mv)
                mi = jnp.where(better, ti, mi)
            ca = jnp.where((kcols == j) & frow, mv, ca)
            ia = jnp.where((kcols == j) & frow, mi, ia)
            return ca, ia, mv, mi

        pv0 = jnp.full((_BM, 1), -_INF, jnp.float32)
        pi0 = jnp.full((_BM, 1), -1, jnp.int32)
        ca, ia, _, _ = jax.lax.fori_loop(
            0, _K, fb_body, (cacc2, iacc2, pv0, pi0))
        return ca, ia

    cacc, iacc = jax.lax.cond(
        jnp.any(frow), fallback, lambda c: c, (cacc, iacc))
    c_ref[...] = cacc
    i_ref[...] = iacc


def kernel(X, k):
    del k
    xt = X.T
    grid = (_N // _BM,)
    c, idx = pl.pallas_call(
        _knn_block_kernel,
        grid=grid,
        in_specs=[
            pl.BlockSpec((_BM, _D), lambda i: (i, 0)),
            pl.BlockSpec((_D, _N), lambda i: (0, 0)),
        ],
        out_specs=[
            pl.BlockSpec((_BM, _K), lambda i: (i, 0)),
            pl.BlockSpec((_BM, _K), lambda i: (i, 0)),
        ],
        out_shape=[
            jax.ShapeDtypeStruct((_N, _K), jnp.float32),
            jax.ShapeDtypeStruct((_N, _K), jnp.int32),
        ],
        scratch_shapes=[pltpu.VMEM((_BM, _N), jnp.float32)],
    )(X, xt)
    return c, idx


# speedup vs baseline: 7.1563x; 1.4055x over previous
import jax
import jax.numpy as jnp
from jax.experimental import pallas as pl
from jax.experimental.pallas import tpu as pltpu

_N = 16384
_D = 64
_K = 32
_BM = 128
_CW = 2048
_GW = 256
_NG = _N // _GW
_GSH = 8
_T = 7
_INF = jnp.inf


def _knn_block_kernel(xb_ref, xt_ref, c_ref, i_ref, dscr_ref):
    i = pl.program_id(0)
    xb = xb_ref[...]
    sq_row = jnp.sum(xb * xb, axis=1, keepdims=True)
    gv_l = [[] for _ in range(_T)]
    gi_l = [[] for _ in range(_T)]
    ng_c = _CW // _GW
    for cc in range(_N // _CW):
        xt_c = xt_ref[:, cc * _CW:(cc + 1) * _CW]
        sq_c = jnp.sum(xt_c * xt_c, axis=0, keepdims=True)
        d = sq_row + sq_c - 2.0 * jnp.dot(
            xb, xt_c, preferred_element_type=jnp.float32)
        colc = jax.lax.broadcasted_iota(jnp.int32, (_BM, _CW), 1) + cc * _CW
        rowc = jax.lax.broadcasted_iota(jnp.int32, (_BM, _CW), 0) + i * _BM
        d = jnp.where(colc == rowc, _INF, d)
        dscr_ref[:, cc * _CW:(cc + 1) * _CW] = d
        m = d.reshape(_BM, ng_c, _GW)
        c3 = colc.reshape(_BM, ng_c, _GW)
        for lvl in range(_T):
            av = jnp.min(m, axis=2)
            ai = jnp.min(jnp.where(m == av[:, :, None], c3, _N), axis=2)
            gv_l[lvl].append(av)
            gi_l[lvl].append(ai)
            if lvl + 1 < _T:
                m = jnp.where(c3 == ai[:, :, None], _INF, m)
    hv = [jnp.concatenate(gv_l[t], axis=1) for t in range(_T)]
    hi = [jnp.concatenate(gi_l[t], axis=1) for t in range(_T)]

    kcols = jax.lax.broadcasted_iota(jnp.int32, (_BM, _K), 1)
    giota = jax.lax.broadcasted_iota(jnp.int32, (_BM, _NG), 1)

    def body(j, carry):
        hv, hi, cacc, iacc = carry
        w = hv[0]
        mv = jnp.min(w, axis=1, keepdims=True)
        gsel = jnp.min(jnp.where(w == mv, giota, _NG), axis=1, keepdims=True)
        onehot = giota == gsel
        mi = jnp.sum(jnp.where(onehot, hi[0], 0), axis=1, keepdims=True)
        cacc = jnp.where(kcols == j, mv, cacc)
        iacc = jnp.where(kcols == j, mi, iacc)
        hv = [jnp.where(onehot, hv[t + 1], hv[t]) for t in range(_T - 1)] + [
            jnp.where(onehot, _INF, hv[_T - 1])]
        hi = [jnp.where(onehot, hi[t + 1], hi[t]) for t in range(_T - 1)] + [
            jnp.where(onehot, _N, hi[_T - 1])]
        return hv, hi, cacc, iacc

    cacc0 = jnp.zeros((_BM, _K), jnp.float32)
    iacc0 = jnp.zeros((_BM, _K), jnp.int32)
    _, _, cacc, iacc = jax.lax.fori_loop(0, _K, body, (hv, hi, cacc0, iacc0))

    giota3 = jax.lax.broadcasted_iota(jnp.int32, (_BM, _K, _NG), 2)
    grp_sel = jax.lax.shift_right_logical(iacc, _GSH)
    cnt_sel = jnp.sum((grp_sel[:, :, None] == giota3).astype(jnp.int32),
                      axis=1)
    frow = jnp.any(cnt_sel >= _T, axis=1, keepdims=True)

    def fallback(carry):
        cacc2, iacc2 = carry

        def fb_body(j, c2):
            ca, ia, pv, pi = c2
            mv = jnp.full((_BM, 1), _INF, jnp.float32)
            mi = jnp.full((_BM, 1), _N, jnp.int32)
            for cc in range(_N // _CW):
                dc = dscr_ref[:, cc * _CW:(cc + 1) * _CW]
                colc = (jax.lax.broadcasted_iota(jnp.int32, (_BM, _CW), 1)
                        + cc * _CW)
                validc = (dc > pv) | ((dc == pv) & (colc > pi))
                tv = jnp.min(jnp.where(validc, dc, _INF),
                             axis=1, keepdims=True)
                ti = jnp.min(jnp.where(validc & (dc == tv), colc, _N),
                             axis=1, keepdims=True)
                better = (tv < mv) | ((tv == mv) & (ti < mi))
                mv = jnp.where(better, tv, mv)
                mi = jnp.where(better, ti, mi)
            ca = jnp.where((kcols == j) & frow, mv, ca)
            ia = jnp.where((kcols == j) & frow, mi, ia)
            return ca, ia, mv, mi

        pv0 = jnp.full((_BM, 1), -_INF, jnp.float32)
        pi0 = jnp.full((_BM, 1), -1, jnp.int32)
        ca, ia, _, _ = jax.lax.fori_loop(
            0, _K, fb_body, (cacc2, iacc2, pv0, pi0))
        return ca, ia

    cacc, iacc = jax.lax.cond(
        jnp.any(frow), fallback, lambda c: c, (cacc, iacc))
    c_ref[...] = cacc
    i_ref[...] = iacc


def kernel(X, k):
    del k
    xt = X.T
    grid = (_N // _BM,)
    c, idx = pl.pallas_call(
        _knn_block_kernel,
        grid=grid,
        in_specs=[
            pl.BlockSpec((_BM, _D), lambda i: (i, 0)),
            pl.BlockSpec((_D, _N), lambda i: (0, 0)),
        ],
        out_specs=[
            pl.BlockSpec((_BM, _K), lambda i: (i, 0)),
            pl.BlockSpec((_BM, _K), lambda i: (i, 0)),
        ],
        out_shape=[
            jax.ShapeDtypeStruct((_N, _K), jnp.float32),
            jax.ShapeDtypeStruct((_N, _K), jnp.int32),
        ],
        scratch_shapes=[pltpu.VMEM((_BM, _N), jnp.float32)],
    )(X, xt)
    return c, idx
